# Initial kernel scaffold; baseline (speedup 1.0000x reference)
import jax
import jax.numpy as jnp
from jax.experimental import pallas as pl


def kernel(x, P_vec, sub_adj, W1, b1, W2, b2):
    raise NotImplementedError("write your pallas kernel here")


# speedup vs baseline: 149.3629x; 149.3629x over previous
import functools

import jax
import jax.numpy as jnp
from jax.experimental import pallas as pl
from jax.experimental.pallas import tpu as pltpu
from jax.experimental.pallas import tpu_sc as plsc

N = 4096
BLK = 512
NB = N // BLK
RW = N + BLK
CPY = N + 8
NUM_WORKERS = 32
ROWS_PER_WORKER = N // NUM_WORKERS


def _unpack_rows(P_pad):
    mesh = plsc.VectorSubcoreMesh(core_axis_name="c", subcore_axis_name="s")
    NBUF = 8
    LOOK = 4

    @functools.partial(
        pl.kernel,
        out_type=jax.ShapeDtypeStruct((N * RW,), jnp.float32),
        mesh=mesh,
        scratch_types=[
            pltpu.VMEM((NBUF * CPY,), jnp.float32),
            pltpu.SemaphoreType.DMA((NBUF,)),
            pltpu.SemaphoreType.DMA((NBUF,)),
        ],
    )
    def k(pvec_hbm, r_hbm, buf, sem_in, sem_out):
        wid = jax.lax.axis_index("c") * 16 + jax.lax.axis_index("s")
        row0 = wid * ROWS_PER_WORKER

        def load_copy(t):
            i = row0 + t
            start = pl.multiple_of(((i * (i + 1)) // 2) & ~7, 8)
            return pltpu.make_async_copy(
                pvec_hbm.at[pl.ds(start, CPY)],
                buf.at[pl.ds(pl.multiple_of((t % NBUF) * CPY, 8), CPY)],
                sem_in.at[t % NBUF],
            )

        def store_copy(t):
            i = row0 + t
            return pltpu.make_async_copy(
                buf.at[pl.ds(pl.multiple_of((t % NBUF) * CPY, 8), CPY)],
                r_hbm.at[pl.ds(pl.multiple_of(i * RW, 8), CPY)],
                sem_out.at[t % NBUF],
            )

        @pl.loop(0, LOOK)
        def _(t):
            load_copy(t).start()

        @pl.loop(0, ROWS_PER_WORKER)
        def _(t):
            u = t + LOOK

            @pl.when(u < ROWS_PER_WORKER)
            def _():
                @pl.when(u >= NBUF)
                def _():
                    store_copy(u - NBUF).wait()

                load_copy(u).start()

            load_copy(t).wait()
            store_copy(t).start()

        @pl.loop(ROWS_PER_WORKER - NBUF, ROWS_PER_WORKER)
        def _(t):
            store_copy(t).wait()

    return k(P_pad).reshape(N, RW)


def _symm_body(r0_ref, r1_ref, sh_ref, s_ref, p_ref, a_ref, d_ref):
    i = pl.program_id(0)
    j = pl.program_id(1)
    cat = jnp.concatenate([r0_ref[...], r1_ref[...]], axis=1)
    sh = sh_ref[...]
    rb = cat[:, 0:BLK]
    for s in range(1, 8):
        rb = jnp.where(sh == s, cat[:, s:s + BLK], rb)
    rt = rb.T
    rows = jax.lax.broadcasted_iota(jnp.int32, (BLK, BLK), 0)
    cols = jax.lax.broadcasted_iota(jnp.int32, (BLK, BLK), 1)
    diag_mix = jnp.where(rows >= cols, rb, rt)
    ps = jnp.where(i > j, rb, jnp.where(i < j, rt, diag_mix))
    pu = jax.nn.sigmoid(ps)
    ab = pu * s_ref[...]
    p_ref[...] = pu
    a_ref[...] = ab
    rs = jnp.sum(ab, axis=1, keepdims=True)

    @pl.when(j == 0)
    def _():
        d_ref[...] = rs + 1.0

    @pl.when(j > 0)
    def _():
        d_ref[...] = d_ref[...] + rs


def _symm_mask_deg(R, shifts, sub_adj):
    return pl.pallas_call(
        _symm_body,
        grid=(NB, NB),
        in_specs=[
            pl.BlockSpec((BLK, BLK), lambda i, j: (jnp.maximum(i, j), jnp.minimum(i, j))),
            pl.BlockSpec((BLK, BLK), lambda i, j: (jnp.maximum(i, j), jnp.minimum(i, j) + 1)),
            pl.BlockSpec((BLK, 1), lambda i, j: (jnp.maximum(i, j), 0)),
            pl.BlockSpec((BLK, BLK), lambda i, j: (i, j)),
        ],
        out_specs=[
            pl.BlockSpec((BLK, BLK), lambda i, j: (i, j)),
            pl.BlockSpec((BLK, BLK), lambda i, j: (i, j)),
            pl.BlockSpec((BLK, 1), lambda i, j: (i, 0)),
        ],
        out_shape=[
            jax.ShapeDtypeStruct((N, N), jnp.float32),
            jax.ShapeDtypeStruct((N, N), jnp.float32),
            jax.ShapeDtypeStruct((N, 1), jnp.float32),
        ],
    )(R, R, shifts, sub_adj)


def _dense_g_body(x_ref, w_ref, d_ref, o_ref):
    h = jnp.dot(
        x_ref[...], w_ref[...],
        preferred_element_type=jnp.float32,
        precision=jax.lax.Precision.HIGHEST,
    )
    o_ref[...] = jax.lax.rsqrt(d_ref[...]) * h


def _dense_g(xin, W, deg):
    f = W.shape[1]
    return pl.pallas_call(
        _dense_g_body,
        grid=(NB,),
        in_specs=[
            pl.BlockSpec((BLK, xin.shape[1]), lambda i: (i, 0)),
            pl.BlockSpec((W.shape[0], f), lambda i: (0, 0)),
            pl.BlockSpec((BLK, 1), lambda i: (i, 0)),
        ],
        out_specs=pl.BlockSpec((BLK, f), lambda i: (i, 0)),
        out_shape=jax.ShapeDtypeStruct((N, f), jnp.float32),
    )(xin, W, deg)


def _make_spmm_body(relu):
    def body(a_ref, g_ref, gi_ref, d_ref, b_ref, o_ref, acc_ref):
        k = pl.program_id(1)
        part = jnp.dot(
            a_ref[...], g_ref[...],
            preferred_element_type=jnp.float32,
            precision=jax.lax.Precision.HIGHEST,
        )

        @pl.when(k == 0)
        def _():
            acc_ref[...] = part

        @pl.when(k > 0)
        def _():
            acc_ref[...] = acc_ref[...] + part

        @pl.when(k == NB - 1)
        def _():
            dinv = jax.lax.rsqrt(d_ref[...])
            y = dinv * (acc_ref[...] + gi_ref[...]) + b_ref[...]
            o_ref[...] = jnp.maximum(y, 0.0) if relu else y

    return body


def _spmm(adj, g, deg, bias2d, relu):
    f = g.shape[1]
    return pl.pallas_call(
        _make_spmm_body(relu),
        grid=(NB, NB),
        in_specs=[
            pl.BlockSpec((BLK, BLK), lambda i, k: (i, k)),
            pl.BlockSpec((BLK, f), lambda i, k: (k, 0)),
            pl.BlockSpec((BLK, f), lambda i, k: (i, 0)),
            pl.BlockSpec((BLK, 1), lambda i, k: (i, 0)),
            pl.BlockSpec((1, f), lambda i, k: (0, 0)),
        ],
        out_specs=pl.BlockSpec((BLK, f), lambda i, k: (i, 0)),
        out_shape=jax.ShapeDtypeStruct((N, f), jnp.float32),
        scratch_shapes=[pltpu.VMEM((BLK, f), jnp.float32)],
    )(adj, g, g, deg, bias2d)


def kernel(x, P_vec, sub_adj, W1, b1, W2, b2):
    P_pad = jnp.concatenate([P_vec, jnp.zeros((8,), P_vec.dtype)])
    rows = jnp.arange(N, dtype=jnp.int32)
    shifts = (((rows * (rows + 1)) // 2) % 8).reshape(N, 1)
    R = _unpack_rows(P_pad)
    P_used, adj, deg = _symm_mask_deg(R, shifts, sub_adj)
    g1 = _dense_g(x, W1, deg)
    z1 = _spmm(adj, g1, deg, b1.reshape(1, -1), relu=True)
    g2 = _dense_g(z1, W2, deg)
    out = _spmm(adj, g2, deg, b2.reshape(1, -1), relu=False)
    return out, P_used


# speedup vs baseline: 181.5393x; 1.2154x over previous
import functools

import jax
import jax.numpy as jnp
from jax.experimental import pallas as pl
from jax.experimental.pallas import tpu as pltpu
from jax.experimental.pallas import tpu_sc as plsc

N = 4096
BLK = 512
NB = N // BLK
RW = N + BLK
CPY = N + 8
NUM_WORKERS = 32
ROWS_PER_WORKER = N // NUM_WORKERS


def _unpack_rows(P_pad):
    mesh = plsc.VectorSubcoreMesh(core_axis_name="c", subcore_axis_name="s")
    NBUF = 8
    LOOK = 4

    @functools.partial(
        pl.kernel,
        out_type=jax.ShapeDtypeStruct((N * RW,), jnp.float32),
        mesh=mesh,
        scratch_types=[
            pltpu.VMEM((NBUF * CPY,), jnp.float32),
            pltpu.SemaphoreType.DMA((NBUF,)),
            pltpu.SemaphoreType.DMA((NBUF,)),
        ],
    )
    def k(pvec_hbm, r_hbm, buf, sem_in, sem_out):
        wid = jax.lax.axis_index("c") * 16 + jax.lax.axis_index("s")
        row0 = wid * ROWS_PER_WORKER

        def load_copy(t):
            i = row0 + t
            start = pl.multiple_of(((i * (i + 1)) // 2) & ~7, 8)
            return pltpu.make_async_copy(
                pvec_hbm.at[pl.ds(start, CPY)],
                buf.at[pl.ds(pl.multiple_of((t % NBUF) * CPY, 8), CPY)],
                sem_in.at[t % NBUF],
            )

        def store_copy(t):
            i = row0 + t
            return pltpu.make_async_copy(
                buf.at[pl.ds(pl.multiple_of((t % NBUF) * CPY, 8), CPY)],
                r_hbm.at[pl.ds(pl.multiple_of(i * RW, 8), CPY)],
                sem_out.at[t % NBUF],
            )

        @pl.loop(0, LOOK)
        def _(t):
            load_copy(t).start()

        @pl.loop(0, ROWS_PER_WORKER)
        def _(t):
            u = t + LOOK

            @pl.when(u < ROWS_PER_WORKER)
            def _():
                @pl.when(u >= NBUF)
                def _():
                    store_copy(u - NBUF).wait()

                load_copy(u).start()

            load_copy(t).wait()
            store_copy(t).start()

        @pl.loop(ROWS_PER_WORKER - NBUF, ROWS_PER_WORKER)
        def _(t):
            store_copy(t).wait()

    return k(P_pad).reshape(N, RW)


def _symm_body(r0_ref, r1_ref, sh_ref, s_ref, p_ref, a_ref, d_ref):
    i = pl.program_id(0)
    j = pl.program_id(1)
    cat = jnp.concatenate([r0_ref[...], r1_ref[...]], axis=1)
    sh = sh_ref[...]
    rb = cat[:, 0:BLK]
    for s in range(1, 8):
        rb = jnp.where(sh == s, cat[:, s:s + BLK], rb)

    def finish(ps):
        pu = jax.nn.sigmoid(ps)
        ab = pu * s_ref[...]
        p_ref[...] = pu
        a_ref[...] = ab.astype(jnp.bfloat16)
        rs = jnp.sum(ab, axis=1, keepdims=True)

        @pl.when(j == 0)
        def _():
            d_ref[...] = rs + 1.0

        @pl.when(j > 0)
        def _():
            d_ref[...] = d_ref[...] + rs

    @pl.when(i > j)
    def _():
        finish(rb)

    @pl.when(i < j)
    def _():
        finish(rb.T)

    @pl.when(i == j)
    def _():
        rows = jax.lax.broadcasted_iota(jnp.int32, (BLK, BLK), 0)
        cols = jax.lax.broadcasted_iota(jnp.int32, (BLK, BLK), 1)
        finish(jnp.where(rows >= cols, rb, rb.T))


def _symm_mask_deg(R, shifts, sub_adj):
    return pl.pallas_call(
        _symm_body,
        grid=(NB, NB),
        in_specs=[
            pl.BlockSpec((BLK, BLK), lambda i, j: (jnp.maximum(i, j), jnp.minimum(i, j))),
            pl.BlockSpec((BLK, BLK), lambda i, j: (jnp.maximum(i, j), jnp.minimum(i, j) + 1)),
            pl.BlockSpec((BLK, 1), lambda i, j: (jnp.maximum(i, j), 0)),
            pl.BlockSpec((BLK, BLK), lambda i, j: (i, j)),
        ],
        out_specs=[
            pl.BlockSpec((BLK, BLK), lambda i, j: (i, j)),
            pl.BlockSpec((BLK, BLK), lambda i, j: (i, j)),
            pl.BlockSpec((BLK, 1), lambda i, j: (i, 0)),
        ],
        out_shape=[
            jax.ShapeDtypeStruct((N, N), jnp.float32),
            jax.ShapeDtypeStruct((N, N), jnp.bfloat16),
            jax.ShapeDtypeStruct((N, 1), jnp.float32),
        ],
    )(R, R, shifts, sub_adj)


def _dense_g_body(x_ref, w_ref, d_ref, o_ref):
    h = jnp.dot(
        x_ref[...], w_ref[...],
        preferred_element_type=jnp.float32,
        precision=jax.lax.Precision.HIGHEST,
    )
    o_ref[...] = jax.lax.rsqrt(d_ref[...]) * h


def _dense_g(xin, W, deg):
    f = W.shape[1]
    return pl.pallas_call(
        _dense_g_body,
        grid=(NB,),
        in_specs=[
            pl.BlockSpec((BLK, xin.shape[1]), lambda i: (i, 0)),
            pl.BlockSpec((W.shape[0], f), lambda i: (0, 0)),
            pl.BlockSpec((BLK, 1), lambda i: (i, 0)),
        ],
        out_specs=pl.BlockSpec((BLK, f), lambda i: (i, 0)),
        out_shape=jax.ShapeDtypeStruct((N, f), jnp.float32),
    )(xin, W, deg)


def _make_spmm_body(relu):
    def body(a_ref, g_ref, gi_ref, d_ref, b_ref, o_ref, acc_ref):
        k = pl.program_id(1)
        part = jnp.dot(
            a_ref[...], g_ref[...].astype(jnp.bfloat16),
            preferred_element_type=jnp.float32,
        )

        @pl.when(k == 0)
        def _():
            acc_ref[...] = part

        @pl.when(k > 0)
        def _():
            acc_ref[...] = acc_ref[...] + part

        @pl.when(k == NB - 1)
        def _():
            dinv = jax.lax.rsqrt(d_ref[...])
            y = dinv * (acc_ref[...] + gi_ref[...]) + b_ref[...]
            o_ref[...] = jnp.maximum(y, 0.0) if relu else y

    return body


def _spmm(adj, g, deg, bias2d, relu):
    f = g.shape[1]
    return pl.pallas_call(
        _make_spmm_body(relu),
        grid=(NB, NB),
        in_specs=[
            pl.BlockSpec((BLK, BLK), lambda i, k: (i, k)),
            pl.BlockSpec((BLK, f), lambda i, k: (k, 0)),
            pl.BlockSpec((BLK, f), lambda i, k: (i, 0)),
            pl.BlockSpec((BLK, 1), lambda i, k: (i, 0)),
            pl.BlockSpec((1, f), lambda i, k: (0, 0)),
        ],
        out_specs=pl.BlockSpec((BLK, f), lambda i, k: (i, 0)),
        out_shape=jax.ShapeDtypeStruct((N, f), jnp.float32),
        scratch_shapes=[pltpu.VMEM((BLK, f), jnp.float32)],
    )(adj, g, g, deg, bias2d)


def kernel(x, P_vec, sub_adj, W1, b1, W2, b2):
    P_pad = jnp.concatenate([P_vec, jnp.zeros((8,), P_vec.dtype)])
    rows = jnp.arange(N, dtype=jnp.int32)
    shifts = (((rows * (rows + 1)) // 2) % 8).reshape(N, 1)
    R = _unpack_rows(P_pad)
    P_used, adj, deg = _symm_mask_deg(R, shifts, sub_adj)
    g1 = _dense_g(x, W1, deg)
    z1 = _spmm(adj, g1, deg, b1.reshape(1, -1), relu=True)
    g2 = _dense_g(z1, W2, deg)
    out = _spmm(adj, g2, deg, b2.reshape(1, -1), relu=False)
    return out, P_used


# speedup vs baseline: 194.7397x; 1.0727x over previous
import functools

import jax
import jax.numpy as jnp
from jax.experimental import pallas as pl
from jax.experimental.pallas import tpu as pltpu
from jax.experimental.pallas import tpu_sc as plsc

N = 4096
BLK = 512
NB = N // BLK
RW = N + BLK
CPY = N + 8
P_SIZE = (N * N - N) // 2 + N
NUM_WORKERS = 32
ROWS_PER_WORKER = N // NUM_WORKERS
T_PER_BUCKET = ROWS_PER_WORKER // NB


def _unpack_rows(P_vec):
    mesh = plsc.VectorSubcoreMesh(core_axis_name="c", subcore_axis_name="s")
    NBUF = 8
    LOOK = 4

    def blen(q):
        return BLK * (q + 1) + 8

    @functools.partial(
        pl.kernel,
        out_type=jax.ShapeDtypeStruct((N * RW,), jnp.float32),
        mesh=mesh,
        scratch_types=[
            pltpu.VMEM((NBUF * CPY,), jnp.float32),
            pltpu.SemaphoreType.DMA((NBUF,)),
            pltpu.SemaphoreType.DMA((NBUF,)),
        ],
    )
    def k(pvec_hbm, r_hbm, buf, sem_in, sem_out):
        wid = jax.lax.axis_index("c") * 16 + jax.lax.axis_index("s")

        def load_copy(t, ln):
            i = t * NUM_WORKERS + wid
            start = ((i * (i + 1)) // 2) & ~7
            start = pl.multiple_of(jnp.minimum(start, P_SIZE - ln), 8)
            return pltpu.make_async_copy(
                pvec_hbm.at[pl.ds(start, ln)],
                buf.at[pl.ds(pl.multiple_of((t % NBUF) * CPY, 8), ln)],
                sem_in.at[t % NBUF],
            )

        def store_copy(t, ln):
            i = t * NUM_WORKERS + wid
            return pltpu.make_async_copy(
                buf.at[pl.ds(pl.multiple_of((t % NBUF) * CPY, 8), ln)],
                r_hbm.at[pl.ds(pl.multiple_of(i * RW, 8), ln)],
                sem_out.at[t % NBUF],
            )

        @pl.loop(0, LOOK)
        def _(t):
            load_copy(t, blen(0)).start()

        for q in range(NB):
            t_lo, t_hi = q * T_PER_BUCKET, (q + 1) * T_PER_BUCKET

            @pl.loop(t_lo, t_hi)
            def _(t, q=q):
                u = t + LOOK

                @pl.when(u < ROWS_PER_WORKER)
                def _():
                    @pl.when(u >= NBUF)
                    def _():
                        v = u - NBUF

                        @pl.when(v >= t_lo)
                        def _():
                            store_copy(v, blen(q)).wait()

                        if q > 0:
                            @pl.when(v < t_lo)
                            def _():
                                store_copy(v, blen(q - 1)).wait()

                    @pl.when(u < t_hi)
                    def _():
                        load_copy(u, blen(q)).start()

                    if q + 1 < NB:
                        @pl.when(u >= t_hi)
                        def _():
                            load_copy(u, blen(q + 1)).start()

                load_copy(t, blen(q)).wait()
                store_copy(t, blen(q)).start()

        @pl.loop(ROWS_PER_WORKER - NBUF, ROWS_PER_WORKER)
        def _(t):
            store_copy(t, blen(NB - 1)).wait()

    return k(P_vec).reshape(N, RW)


def _symm_body(r0_ref, r1_ref, s_ref, p_ref, a_ref, d_ref):
    i = pl.program_id(0)
    j = pl.program_id(1)
    a = jnp.maximum(i, j)
    rg = a * BLK + jax.lax.broadcasted_iota(jnp.int32, (BLK, 1), 0)
    sh = ((rg * (rg + 1)) // 2) & 7
    sh = jnp.where(rg == N - 1, 8, sh)
    cat = jnp.concatenate([r0_ref[...], r1_ref[...]], axis=1)
    rb = cat[:, 0:BLK]
    for s in range(1, 9):
        rb = jnp.where(sh == s, cat[:, s:s + BLK], rb)

    def finish(ps):
        pu = jax.nn.sigmoid(ps)
        ab = pu * s_ref[...]
        p_ref[...] = pu
        a_ref[...] = ab.astype(jnp.bfloat16)
        rs = jnp.sum(ab, axis=1, keepdims=True)

        @pl.when(j == 0)
        def _():
            d_ref[...] = rs + 1.0

        @pl.when(j > 0)
        def _():
            d_ref[...] = d_ref[...] + rs

    @pl.when(i > j)
    def _():
        finish(rb)

    @pl.when(i < j)
    def _():
        finish(rb.T)

    @pl.when(i == j)
    def _():
        rows = jax.lax.broadcasted_iota(jnp.int32, (BLK, BLK), 0)
        cols = jax.lax.broadcasted_iota(jnp.int32, (BLK, BLK), 1)
        finish(jnp.where(rows >= cols, rb, rb.T))


def _symm_mask_deg(R, sub_adj):
    return pl.pallas_call(
        _symm_body,
        grid=(NB, NB),
        in_specs=[
            pl.BlockSpec((BLK, BLK), lambda i, j: (jnp.maximum(i, j), jnp.minimum(i, j))),
            pl.BlockSpec((BLK, BLK), lambda i, j: (jnp.maximum(i, j), jnp.minimum(i, j) + 1)),
            pl.BlockSpec((BLK, BLK), lambda i, j: (i, j)),
        ],
        out_specs=[
            pl.BlockSpec((BLK, BLK), lambda i, j: (i, j)),
            pl.BlockSpec((BLK, BLK), lambda i, j: (i, j)),
            pl.BlockSpec((BLK, 1), lambda i, j: (i, 0)),
        ],
        out_shape=[
            jax.ShapeDtypeStruct((N, N), jnp.float32),
            jax.ShapeDtypeStruct((N, N), jnp.bfloat16),
            jax.ShapeDtypeStruct((N, 1), jnp.float32),
        ],
    )(R, R, sub_adj)


def _matmul_body(x_ref, w_ref, o_ref):
    o_ref[...] = jnp.dot(
        x_ref[...], w_ref[...],
        preferred_element_type=jnp.float32,
        precision=jax.lax.Precision.HIGHEST,
    )


def _matmul(xin, W):
    f = W.shape[1]
    return pl.pallas_call(
        _matmul_body,
        grid=(NB,),
        in_specs=[
            pl.BlockSpec((BLK, xin.shape[1]), lambda i: (i, 0)),
            pl.BlockSpec((W.shape[0], f), lambda i: (0, 0)),
        ],
        out_specs=pl.BlockSpec((BLK, f), lambda i: (i, 0)),
        out_shape=jax.ShapeDtypeStruct((N, f), jnp.float32),
    )(xin, W)


def _make_spmm_body(relu, with_w):
    def transform(h, dk, w_ref):
        if with_w:
            h = jnp.dot(
                h, w_ref[...],
                preferred_element_type=jnp.float32,
                precision=jax.lax.Precision.HIGHEST,
            )
        return jax.lax.rsqrt(dk) * h

    def body(a_ref, g_ref, gi_ref, dk_ref, di_ref, b_ref, *rest):
        if with_w:
            w_ref, o_ref, acc_ref = rest
        else:
            w_ref, (o_ref, acc_ref) = None, rest
        k = pl.program_id(1)
        gk = transform(g_ref[...], dk_ref[...], w_ref)
        part = jnp.dot(
            a_ref[...], gk.astype(jnp.bfloat16),
            preferred_element_type=jnp.float32,
        )

        @pl.when(k == 0)
        def _():
            acc_ref[...] = part

        @pl.when(k > 0)
        def _():
            acc_ref[...] = acc_ref[...] + part

        @pl.when(k == NB - 1)
        def _():
            dinv = jax.lax.rsqrt(di_ref[...])
            gi = transform(gi_ref[...], di_ref[...], w_ref)
            y = dinv * (acc_ref[...] + gi) + b_ref[...]
            o_ref[...] = jnp.maximum(y, 0.0) if relu else y

    return body


def _spmm(adj, h, deg, bias2d, W=None, relu=False):
    fin = h.shape[1]
    f = W.shape[1] if W is not None else fin
    ins = [
        pl.BlockSpec((BLK, BLK), lambda i, k: (i, k)),
        pl.BlockSpec((BLK, fin), lambda i, k: (k, 0)),
        pl.BlockSpec((BLK, fin), lambda i, k: (i, 0)),
        pl.BlockSpec((BLK, 1), lambda i, k: (k, 0)),
        pl.BlockSpec((BLK, 1), lambda i, k: (i, 0)),
        pl.BlockSpec((1, f), lambda i, k: (0, 0)),
    ]
    args = [adj, h, h, deg, deg, bias2d]
    if W is not None:
        ins.append(pl.BlockSpec((W.shape[0], f), lambda i, k: (0, 0)))
        args.append(W)
    return pl.pallas_call(
        _make_spmm_body(relu, W is not None),
        grid=(NB, NB),
        in_specs=ins,
        out_specs=pl.BlockSpec((BLK, f), lambda i, k: (i, 0)),
        out_shape=jax.ShapeDtypeStruct((N, f), jnp.float32),
        scratch_shapes=[pltpu.VMEM((BLK, f), jnp.float32)],
    )(*args)


def kernel(x, P_vec, sub_adj, W1, b1, W2, b2):
    R = _unpack_rows(P_vec)
    h1 = _matmul(x, W1)
    P_used, adj, deg = _symm_mask_deg(R, sub_adj)
    z1 = _spmm(adj, h1, deg, b1.reshape(1, -1), relu=True)
    out = _spmm(adj, z1, deg, b2.reshape(1, -1), W=W2)
    return out, P_used


# speedup vs baseline: 215.2532x; 1.1053x over previous
import functools

import jax
import jax.numpy as jnp
from jax.experimental import pallas as pl
from jax.experimental.pallas import tpu as pltpu
from jax.experimental.pallas import tpu_sc as plsc

N = 4096
BLK = 512
NB = N // BLK
RW = N + BLK
P_SIZE = (N * N - N) // 2 + N
NUM_WORKERS = 32


ROWS_PER_WORKER = N // NUM_WORKERS
T_PER_BUCKET = ROWS_PER_WORKER // NB
CPY = N + 8


def _unpack_rows(P_vec):
    mesh = plsc.VectorSubcoreMesh(core_axis_name="c", subcore_axis_name="s")
    NBUF = 8
    LOOK = 4

    def blen(q):
        return BLK * (q + 1) + 8

    @functools.partial(
        pl.kernel,
        out_type=jax.ShapeDtypeStruct((N * RW,), jnp.float32),
        mesh=mesh,
        scratch_types=[
            pltpu.VMEM((NBUF * CPY,), jnp.float32),
            pltpu.SemaphoreType.DMA((NBUF,)),
            pltpu.SemaphoreType.DMA((NBUF,)),
        ],
    )
    def k(pvec_hbm, r_hbm, buf, sem_in, sem_out):
        wid = jax.lax.axis_index("c") * 16 + jax.lax.axis_index("s")

        def load_copy(t, ln):
            i = t * NUM_WORKERS + wid
            start = ((i * (i + 1)) // 2) & ~7
            start = pl.multiple_of(jnp.minimum(start, P_SIZE - ln), 8)
            return pltpu.make_async_copy(
                pvec_hbm.at[pl.ds(start, ln)],
                buf.at[pl.ds(pl.multiple_of((t % NBUF) * CPY, 8), ln)],
                sem_in.at[t % NBUF],
            )

        def store_copy(t, ln):
            i = t * NUM_WORKERS + wid
            return pltpu.make_async_copy(
                buf.at[pl.ds(pl.multiple_of((t % NBUF) * CPY, 8), ln)],
                r_hbm.at[pl.ds(pl.multiple_of(i * RW, 8), ln)],
                sem_out.at[t % NBUF],
            )

        @pl.loop(0, LOOK)
        def _(t):
            load_copy(t, blen(0)).start()

        for q in range(NB):
            t_lo, t_hi = q * T_PER_BUCKET, (q + 1) * T_PER_BUCKET

            @pl.loop(t_lo, t_hi)
            def _(t, q=q):
                u = t + LOOK

                @pl.when(u < ROWS_PER_WORKER)
                def _():
                    @pl.when(u >= NBUF)
                    def _():
                        v = u - NBUF

                        @pl.when(v >= t_lo)
                        def _():
                            store_copy(v, blen(q)).wait()

                        if q > 0:
                            @pl.when(v < t_lo)
                            def _():
                                store_copy(v, blen(q - 1)).wait()

                    @pl.when(u < t_hi)
                    def _():
                        load_copy(u, blen(q)).start()

                    if q + 1 < NB:
                        @pl.when(u >= t_hi)
                        def _():
                            load_copy(u, blen(q + 1)).start()

                load_copy(t, blen(q)).wait()
                store_copy(t, blen(q)).start()

        @pl.loop(ROWS_PER_WORKER - NBUF, ROWS_PER_WORKER)
        def _(t):
            store_copy(t, blen(NB - 1)).wait()

    return k(P_vec).reshape(N, RW)


def _symm_body(r0_ref, r1_ref, s_ref, p_ref, a_ref, d_ref):
    i = pl.program_id(0)
    j = pl.program_id(1)
    a = jnp.maximum(i, j)
    rg = a * BLK + jax.lax.broadcasted_iota(jnp.int32, (BLK, 1), 0)
    sh = ((rg * (rg + 1)) // 2) & 7
    sh = jnp.where(rg == N - 1, 8, sh)
    cat = jnp.concatenate([r0_ref[...], r1_ref[...]], axis=1)
    rb = cat[:, 0:BLK]
    for s in range(1, 9):
        rb = jnp.where(sh == s, cat[:, s:s + BLK], rb)

    def finish(ps):
        pu = 0.5 * jnp.tanh(0.5 * ps) + 0.5
        ab = pu * s_ref[...]
        p_ref[...] = pu
        a_ref[...] = ab.astype(jnp.bfloat16)
        rs = jnp.sum(ab, axis=1, keepdims=True)

        @pl.when(j == 0)
        def _():
            d_ref[...] = rs + 1.0

        @pl.when(j > 0)
        def _():
            d_ref[...] = d_ref[...] + rs

    @pl.when(i > j)
    def _():
        finish(rb)

    @pl.when(i < j)
    def _():
        finish(rb.T)

    @pl.when(i == j)
    def _():
        rows = jax.lax.broadcasted_iota(jnp.int32, (BLK, BLK), 0)
        cols = jax.lax.broadcasted_iota(jnp.int32, (BLK, BLK), 1)
        finish(jnp.where(rows >= cols, rb, rb.T))


def _symm_mask_deg(R, sub_adj):
    return pl.pallas_call(
        _symm_body,
        grid=(NB, NB),
        in_specs=[
            pl.BlockSpec((BLK, BLK), lambda i, j: (jnp.maximum(i, j), jnp.minimum(i, j))),
            pl.BlockSpec((BLK, BLK), lambda i, j: (jnp.maximum(i, j), jnp.minimum(i, j) + 1)),
            pl.BlockSpec((BLK, BLK), lambda i, j: (i, j)),
        ],
        out_specs=[
            pl.BlockSpec((BLK, BLK), lambda i, j: (i, j)),
            pl.BlockSpec((BLK, BLK), lambda i, j: (i, j)),
            pl.BlockSpec((BLK, 1), lambda i, j: (i, 0)),
        ],
        out_shape=[
            jax.ShapeDtypeStruct((N, N), jnp.float32),
            jax.ShapeDtypeStruct((N, N), jnp.bfloat16),
            jax.ShapeDtypeStruct((N, 1), jnp.float32),
        ],
    )(R, R, sub_adj)


def _matmul_body(x_ref, w_ref, o_ref):
    o_ref[...] = jnp.dot(
        x_ref[...], w_ref[...],
        preferred_element_type=jnp.float32,
        precision=jax.lax.Precision.HIGHEST,
    )


def _matmul(xin, W):
    f = W.shape[1]
    return pl.pallas_call(
        _matmul_body,
        grid=(NB,),
        in_specs=[
            pl.BlockSpec((BLK, xin.shape[1]), lambda i: (i, 0)),
            pl.BlockSpec((W.shape[0], f), lambda i: (0, 0)),
        ],
        out_specs=pl.BlockSpec((BLK, f), lambda i: (i, 0)),
        out_shape=jax.ShapeDtypeStruct((N, f), jnp.float32),
    )(xin, W)


BLK_K = 1024
NBK = N // BLK_K


def _make_spmm_body(relu):
    def body(a_ref, g_ref, gi_ref, dk_ref, di_ref, b_ref, o_ref, acc_ref):
        k = pl.program_id(1)
        gk = (jax.lax.rsqrt(dk_ref[...]) * g_ref[...]).astype(jnp.bfloat16)
        part = jnp.dot(a_ref[...], gk, preferred_element_type=jnp.float32)

        @pl.when(k == 0)
        def _():
            acc_ref[...] = part

        @pl.when(k > 0)
        def _():
            acc_ref[...] = acc_ref[...] + part

        @pl.when(k == NBK - 1)
        def _():
            dinv = jax.lax.rsqrt(di_ref[...])
            y = dinv * (acc_ref[...] + dinv * gi_ref[...]) + b_ref[...]
            o_ref[...] = jnp.maximum(y, 0.0) if relu else y

    return body


def _spmm(adj, h, deg, bias2d, relu=False):
    f = h.shape[1]
    return pl.pallas_call(
        _make_spmm_body(relu),
        grid=(NB, NBK),
        in_specs=[
            pl.BlockSpec((BLK, BLK_K), lambda i, k: (i, k)),
            pl.BlockSpec((BLK_K, f), lambda i, k: (k, 0)),
            pl.BlockSpec((BLK, f), lambda i, k: (i, 0)),
            pl.BlockSpec((BLK_K, 1), lambda i, k: (k, 0)),
            pl.BlockSpec((BLK, 1), lambda i, k: (i, 0)),
            pl.BlockSpec((1, f), lambda i, k: (0, 0)),
        ],
        out_specs=pl.BlockSpec((BLK, f), lambda i, k: (i, 0)),
        out_shape=jax.ShapeDtypeStruct((N, f), jnp.float32),
        scratch_shapes=[pltpu.VMEM((BLK, f), jnp.float32)],
    )(adj, h, h, deg, deg, bias2d)


def kernel(x, P_vec, sub_adj, W1, b1, W2, b2):
    R = _unpack_rows(P_vec)
    h1 = _matmul(x, W1)
    P_used, adj, deg = _symm_mask_deg(R, sub_adj)
    z1 = _spmm(adj, h1, deg, b1.reshape(1, -1), relu=True)
    h2 = _matmul(z1, W2)
    out = _spmm(adj, h2, deg, b2.reshape(1, -1))
    return out, P_used


# speedup vs baseline: 261.2534x; 1.2137x over previous
import functools

import jax
import jax.numpy as jnp
from jax.experimental import pallas as pl
from jax.experimental.pallas import tpu as pltpu
from jax.experimental.pallas import tpu_sc as plsc

N = 4096
BLK = 512
NB = N // BLK
RW = N + BLK
P_SIZE = (N * N - N) // 2 + N
NUM_WORKERS = 32


ROWS_PER_WORKER = N // NUM_WORKERS
T_PER_BUCKET = ROWS_PER_WORKER // NB
CPY = N + 8


def _unpack_rows(P_vec):
    mesh = plsc.VectorSubcoreMesh(core_axis_name="c", subcore_axis_name="s")
    NBUF = 8
    LOOK = 4

    def blen(q):
        return BLK * (q + 1) + 8

    @functools.partial(
        pl.kernel,
        out_type=jax.ShapeDtypeStruct((N * RW,), jnp.float32),
        mesh=mesh,
        scratch_types=[
            pltpu.VMEM((NBUF * CPY,), jnp.float32),
            pltpu.SemaphoreType.DMA((NBUF,)),
            pltpu.SemaphoreType.DMA((NBUF,)),
        ],
    )
    def k(pvec_hbm, r_hbm, buf, sem_in, sem_out):
        wid = jax.lax.axis_index("c") * 16 + jax.lax.axis_index("s")

        def load_copy(t, ln):
            i = t * NUM_WORKERS + wid
            start = ((i * (i + 1)) // 2) & ~7
            start = pl.multiple_of(jnp.minimum(start, P_SIZE - ln), 8)
            return pltpu.make_async_copy(
                pvec_hbm.at[pl.ds(start, ln)],
                buf.at[pl.ds(pl.multiple_of((t % NBUF) * CPY, 8), ln)],
                sem_in.at[t % NBUF],
            )

        def store_copy(t, ln):
            i = t * NUM_WORKERS + wid
            return pltpu.make_async_copy(
                buf.at[pl.ds(pl.multiple_of((t % NBUF) * CPY, 8), ln)],
                r_hbm.at[pl.ds(pl.multiple_of(i * RW, 8), ln)],
                sem_out.at[t % NBUF],
            )

        @pl.loop(0, LOOK)
        def _(t):
            load_copy(t, blen(0)).start()

        for q in range(NB):
            t_lo, t_hi = q * T_PER_BUCKET, (q + 1) * T_PER_BUCKET

            @pl.loop(t_lo, t_hi)
            def _(t, q=q):
                u = t + LOOK

                @pl.when(u < ROWS_PER_WORKER)
                def _():
                    @pl.when(u >= NBUF)
                    def _():
                        v = u - NBUF

                        @pl.when(v >= t_lo)
                        def _():
                            store_copy(v, blen(q)).wait()

                        if q > 0:
                            @pl.when(v < t_lo)
                            def _():
                                store_copy(v, blen(q - 1)).wait()

                    @pl.when(u < t_hi)
                    def _():
                        load_copy(u, blen(q)).start()

                    if q + 1 < NB:
                        @pl.when(u >= t_hi)
                        def _():
                            load_copy(u, blen(q + 1)).start()

                load_copy(t, blen(q)).wait()
                store_copy(t, blen(q)).start()

        @pl.loop(ROWS_PER_WORKER - NBUF, ROWS_PER_WORKER)
        def _(t):
            store_copy(t, blen(NB - 1)).wait()

    return k(P_vec).reshape(N, RW)


_TRI = [(a, b) for a in range(NB) for b in range(a + 1)]
NTRI = len(_TRI)


def _symm_body(idx_ref, r0_ref, r1_ref, s_ref, p_hbm, adj_hbm, d_ref,
               pu_s, put_s, abf_s, abt_s, dscr, sem):
    k = pl.program_id(0)
    av = idx_ref[0, k]
    bv = idx_ref[1, k]

    def copies(m, slot):
        am = idx_ref[0, m]
        bm = idx_ref[1, m]
        rows_a = pl.ds(pl.multiple_of(am * BLK, BLK), BLK)
        cols_b = pl.ds(pl.multiple_of(bm * BLK, BLK), BLK)
        rows_b = pl.ds(pl.multiple_of(bm * BLK, BLK), BLK)
        cols_a = pl.ds(pl.multiple_of(am * BLK, BLK), BLK)
        mk = pltpu.make_async_copy
        return (
            mk(pu_s.at[slot], p_hbm.at[rows_a, cols_b], sem.at[slot]),
            mk(abf_s.at[slot], adj_hbm.at[rows_a, cols_b], sem.at[slot]),
            mk(put_s.at[slot], p_hbm.at[rows_b, cols_a], sem.at[slot]),
            mk(abt_s.at[slot], adj_hbm.at[rows_b, cols_a], sem.at[slot]),
        )

    def wait_step(m, slot):
        c_p, c_a, c_pt, c_at = copies(m, slot)
        c_p.wait()
        c_a.wait()

        @pl.when(idx_ref[0, m] > idx_ref[1, m])
        def _():
            c_pt.wait()
            c_at.wait()

    def run(slot):
        @pl.when(k >= 2)
        def _():
            wait_step(k - 2, slot)

        @pl.when(k == 0)
        def _():
            dscr[...] = jnp.zeros((NB, BLK, 1), jnp.float32)

        rg = av * BLK + jax.lax.broadcasted_iota(jnp.int32, (BLK, 1), 0)
        sh = ((rg * (rg + 1)) // 2) & 7
        sh = jnp.where(rg == N - 1, 8, sh)
        cat = jnp.concatenate([r0_ref[...], r1_ref[...]], axis=1)
        rb = cat[:, 0:BLK]
        for s in range(1, 9):
            rb = jnp.where(sh == s, cat[:, s:s + BLK], rb)

        c_p, c_a, c_pt, c_at = copies(k, slot)

        @pl.when(av > bv)
        def _():
            pu = 0.5 * jnp.tanh(0.5 * rb) + 0.5
            ab = pu * s_ref[...]
            abt = ab.T
            pu_s[slot] = pu
            put_s[slot] = pu.T
            abf_s[slot] = ab.astype(jnp.bfloat16)
            abt_s[slot] = abt.astype(jnp.bfloat16)
            dscr[av] = dscr[av] + jnp.sum(ab, axis=1, keepdims=True)
            dscr[bv] = dscr[bv] + jnp.sum(abt, axis=1, keepdims=True)
            c_p.start()
            c_a.start()
            c_pt.start()
            c_at.start()

        @pl.when(av == bv)
        def _():
            rows = jax.lax.broadcasted_iota(jnp.int32, (BLK, BLK), 0)
            cols = jax.lax.broadcasted_iota(jnp.int32, (BLK, BLK), 1)
            mix = jnp.where(rows >= cols, rb, rb.T)
            pu = 0.5 * jnp.tanh(0.5 * mix) + 0.5
            ab = pu * s_ref[...]
            pu_s[slot] = pu
            abf_s[slot] = ab.astype(jnp.bfloat16)
            dscr[av] = dscr[av] + jnp.sum(ab, axis=1, keepdims=True)
            c_p.start()
            c_a.start()

        @pl.when(k == NTRI - 1)
        def _():
            for band in range(NB):
                d_ref[pl.ds(band * BLK, BLK), :] = dscr[band] + 1.0
            wait_step(k - 1, 1 - slot)
            wait_step(k, slot)

    @pl.when(k % 2 == 0)
    def _():
        run(0)

    @pl.when(k % 2 == 1)
    def _():
        run(1)


def _symm_mask_deg(R, sub_adj):
    idx = jnp.asarray([[a for a, _ in _TRI], [b for _, b in _TRI]], jnp.int32)
    grid_spec = pltpu.PrefetchScalarGridSpec(
        num_scalar_prefetch=1,
        grid=(NTRI,),
        in_specs=[
            pl.BlockSpec((BLK, BLK), lambda k, idx: (idx[0, k], idx[1, k])),
            pl.BlockSpec((BLK, BLK), lambda k, idx: (idx[0, k], idx[1, k] + 1)),
            pl.BlockSpec((BLK, BLK), lambda k, idx: (idx[0, k], idx[1, k])),
        ],
        out_specs=[
            pl.BlockSpec(memory_space=pltpu.MemorySpace.HBM),
            pl.BlockSpec(memory_space=pltpu.MemorySpace.HBM),
            pl.BlockSpec((N, 1), lambda k, idx: (0, 0)),
        ],
        scratch_shapes=[
            pltpu.VMEM((2, BLK, BLK), jnp.float32),
            pltpu.VMEM((2, BLK, BLK), jnp.float32),
            pltpu.VMEM((2, BLK, BLK), jnp.bfloat16),
            pltpu.VMEM((2, BLK, BLK), jnp.bfloat16),
            pltpu.VMEM((NB, BLK, 1), jnp.float32),
            pltpu.SemaphoreType.DMA((2,)),
        ],
    )
    return pl.pallas_call(
        _symm_body,
        grid_spec=grid_spec,
        out_shape=[
            jax.ShapeDtypeStruct((N, N), jnp.float32),
            jax.ShapeDtypeStruct((N, N), jnp.bfloat16),
            jax.ShapeDtypeStruct((N, 1), jnp.float32),
        ],
    )(idx, R, R, sub_adj)


def _matmul_body(x_ref, w_ref, o_ref):
    o_ref[...] = jnp.dot(
        x_ref[...], w_ref[...],
        preferred_element_type=jnp.float32,
        precision=jax.lax.Precision.HIGHEST,
    )


def _matmul(xin, W):
    f = W.shape[1]
    return pl.pallas_call(
        _matmul_body,
        grid=(NB,),
        in_specs=[
            pl.BlockSpec((BLK, xin.shape[1]), lambda i: (i, 0)),
            pl.BlockSpec((W.shape[0], f), lambda i: (0, 0)),
        ],
        out_specs=pl.BlockSpec((BLK, f), lambda i: (i, 0)),
        out_shape=jax.ShapeDtypeStruct((N, f), jnp.float32),
    )(xin, W)


BLK_K = 1024
NBK = N // BLK_K


def _make_spmm_body(relu):
    def body(a_ref, g_ref, gi_ref, dk_ref, di_ref, b_ref, o_ref, acc_ref):
        k = pl.program_id(1)
        gk = (jax.lax.rsqrt(dk_ref[...]) * g_ref[...]).astype(jnp.bfloat16)
        part = jnp.dot(a_ref[...], gk, preferred_element_type=jnp.float32)

        @pl.when(k == 0)
        def _():
            acc_ref[...] = part

        @pl.when(k > 0)
        def _():
            acc_ref[...] = acc_ref[...] + part

        @pl.when(k == NBK - 1)
        def _():
            dinv = jax.lax.rsqrt(di_ref[...])
            y = dinv * (acc_ref[...] + dinv * gi_ref[...]) + b_ref[...]
            o_ref[...] = jnp.maximum(y, 0.0) if relu else y

    return body


def _spmm(adj, h, deg, bias2d, relu=False):
    f = h.shape[1]
    return pl.pallas_call(
        _make_spmm_body(relu),
        grid=(NB, NBK),
        in_specs=[
            pl.BlockSpec((BLK, BLK_K), lambda i, k: (i, k)),
            pl.BlockSpec((BLK_K, f), lambda i, k: (k, 0)),
            pl.BlockSpec((BLK, f), lambda i, k: (i, 0)),
            pl.BlockSpec((BLK_K, 1), lambda i, k: (k, 0)),
            pl.BlockSpec((BLK, 1), lambda i, k: (i, 0)),
            pl.BlockSpec((1, f), lambda i, k: (0, 0)),
        ],
        out_specs=pl.BlockSpec((BLK, f), lambda i, k: (i, 0)),
        out_shape=jax.ShapeDtypeStruct((N, f), jnp.float32),
        scratch_shapes=[pltpu.VMEM((BLK, f), jnp.float32)],
    )(adj, h, h, deg, deg, bias2d)


def kernel(x, P_vec, sub_adj, W1, b1, W2, b2):
    R = _unpack_rows(P_vec)
    h1 = _matmul(x, W1)
    P_used, adj, deg = _symm_mask_deg(R, sub_adj)
    z1 = _spmm(adj, h1, deg, b1.reshape(1, -1), relu=True)
    h2 = _matmul(z1, W2)
    out = _spmm(adj, h2, deg, b2.reshape(1, -1))
    return out, P_used
